# baseline (speedup 1.0000x reference)
import functools

import jax
import jax.numpy as jnp
from jax import lax
from jax.experimental import pallas as pl
from jax.experimental.pallas import tpu as pltpu
from jax.experimental.pallas import tpu_sc as plsc

NN = 10000
EE = 320000
DD = 128
HH = 128

NW = 32
RPW = 320
LAST_R = NN - (NW - 1) * RPW
EPW = EE // NW
GC = 80
SCC = 4000
GB = 128

_mesh = plsc.VectorSubcoreMesh(core_axis_name="c", subcore_axis_name="s")


def _tables_body(relu_in, x_ref, w_ref, bias_ref, a_ref, b_ref):
    x = x_ref[...]
    if relu_in:
        x = jnp.maximum(x, 0.0)
    wt = w_ref[:DD, :]
    wb = w_ref[DD:, :]
    a_ref[...] = (
        jnp.dot(x, wt - wb, preferred_element_type=jnp.float32,
                precision=lax.Precision.HIGHEST)
        + bias_ref[...]
    )
    b_ref[...] = jnp.dot(x, wb, preferred_element_type=jnp.float32,
                         precision=lax.Precision.HIGHEST)


def _tables(x, w, bias, relu_in):
    blk = 1000
    return pl.pallas_call(
        functools.partial(_tables_body, relu_in),
        grid=(NN // blk,),
        in_specs=[
            pl.BlockSpec((blk, DD), lambda i: (i, 0)),
            pl.BlockSpec((2 * DD, HH), lambda i: (0, 0)),
            pl.BlockSpec((1, HH), lambda i: (0, 0)),
        ],
        out_specs=[
            pl.BlockSpec((blk, HH), lambda i: (i, 0)),
            pl.BlockSpec((blk, HH), lambda i: (i, 0)),
        ],
        out_shape=[
            jax.ShapeDtypeStruct((NN, HH), jnp.float32),
            jax.ShapeDtypeStruct((NN, HH), jnp.float32),
        ],
    )(x, w, bias)


def _zmat_body(p_ref, w_ref, bias_ref, z_ref):
    z_ref[...] = (
        jnp.dot(p_ref[...], w_ref[...], preferred_element_type=jnp.float32,
                precision=lax.Precision.HIGHEST)
        + bias_ref[...]
    )


def _zmat(pre, w, bias):
    blk = 2000
    return pl.pallas_call(
        _zmat_body,
        grid=(EE // blk,),
        in_specs=[
            pl.BlockSpec((blk, HH), lambda i: (i, 0)),
            pl.BlockSpec((HH, HH), lambda i: (0, 0)),
            pl.BlockSpec((1, HH), lambda i: (0, 0)),
        ],
        out_specs=pl.BlockSpec((blk, HH), lambda i: (i, 0)),
        out_shape=jax.ShapeDtypeStruct((EE, HH), jnp.float32),
    )(pre, w, bias)


@functools.partial(
    pl.kernel,
    out_type=jax.ShapeDtypeStruct((EE, HH), jnp.float32),
    mesh=_mesh,
    scratch_types=[
        pltpu.VMEM((GC,), jnp.int32),
        pltpu.VMEM((GC,), jnp.int32),
        pltpu.VMEM((GC, HH), jnp.float32),
        pltpu.VMEM((GC, HH), jnp.float32),
        pltpu.SemaphoreType.DMA,
    ],
)
def _gather_sc(a_hbm, b_hbm, src_hbm, dst_hbm, pre_hbm,
               dstv, srcv, rowsa, rowsb, sem):
    wid = lax.axis_index("s") * 2 + lax.axis_index("c")
    ebase = wid * EPW

    def chunk(i, carry):
        base = ebase + i * GC
        pltpu.sync_copy(dst_hbm.at[pl.ds(base, GC)], dstv)
        pltpu.sync_copy(src_hbm.at[pl.ds(base, GC)], srcv)
        cpa = pltpu.async_copy(a_hbm.at[dstv], rowsa, sem)
        cpb = pltpu.async_copy(b_hbm.at[srcv], rowsb, sem)
        cpa.wait()
        cpb.wait()

        def row(r, c):
            for j in range(HH // 16):
                s = pl.ds(j * 16, 16)
                rowsa[r, s] = jnp.maximum(rowsa[r, s] + rowsb[r, s], 0.0)
            return c

        lax.fori_loop(0, GC, row, 0)
        pltpu.sync_copy(rowsa, pre_hbm.at[pl.ds(base, GC)])
        return carry

    lax.fori_loop(0, EPW // GC, chunk, 0)


@functools.partial(
    pl.kernel,
    out_type=jax.ShapeDtypeStruct((NN, HH), jnp.float32),
    mesh=_mesh,
    scratch_types=[
        pltpu.VMEM((SCC,), jnp.int32),
        pltpu.VMEM((SCC + 16,), jnp.int32),
        pltpu.VMEM((SCC + 16,), jnp.int32),
        pltpu.VMEM((GB, HH), jnp.float32),
        pltpu.VMEM((RPW, HH), jnp.float32),
        pltpu.SemaphoreType.DMA,
    ],
)
def _scatter_sc(z_hbm, dst_hbm, out_hbm,
                dstv, eidl, dlocl, rows, acc, sem):
    wid = lax.axis_index("s") * 2 + lax.axis_index("c")
    lo = wid * RPW
    neg = jnp.full((16,), -jnp.inf, jnp.float32)
    zero_i = jnp.zeros((16,), jnp.int32)

    def initrow(r, c):
        for j in range(HH // 16):
            acc[r, pl.ds(j * 16, 16)] = neg
        return c

    lax.fori_loop(0, RPW, initrow, 0)

    def initlist(i, c):
        eidl[pl.ds(i * 16, 16)] = zero_i
        return c

    lax.fori_loop(0, (SCC + 16) // 16, initlist, 0)

    iota = lax.iota(jnp.int32, 16)
    powvec = (jnp.int32(1) << iota) + jnp.int32(1 << 20)

    def _lanesum(s):
        for k in (1, 2, 4, 8):
            idx = jnp.maximum(iota - k, 0)
            g = lax.gather(
                s, idx[:, None],
                lax.GatherDimensionNumbers(offset_dims=(),
                                           collapsed_slice_dims=(0,),
                                           start_index_map=(0,)),
                (1,), mode=lax.GatherScatterMode.PROMISE_IN_BOUNDS)
            s = s + jnp.where(iota >= k, g, 0)
        return s[15]

    def chunk(ci, carry):
        cbase = ci * SCC
        pltpu.sync_copy(dst_hbm.at[pl.ds(cbase, SCC)], dstv)

        def group(g, st):
            off, pend_e, pend_d = st
            goff = g * 16
            d = dstv[pl.ds(goff, 16)]
            msk = (d >= lo) & (d < lo + RPW)
            tot = _lanesum(jnp.where(msk, powvec, 0))
            cnt = tot >> 20
            bw0 = tot & 0xFFFF

            def match(_k, mst):
                bw, off2, pe, pd = mst
                lb = bw & (-bw)
                l = (jnp.where((lb & 0xFF00) != 0, 8, 0)
                     + jnp.where((lb & 0xF0F0) != 0, 4, 0)
                     + jnp.where((lb & 0xCCCC) != 0, 2, 0)
                     + jnp.where((lb & 0xAAAA) != 0, 1, 0))
                eid = cbase + goff + l
                dl = dstv[pl.ds(goff + l, 16)][0] - lo
                im = iota == (off2 & 15)
                pe = jnp.where(im, eid, pe)
                pd = jnp.where(im, dl, pd)
                woff = off2 - (off2 & 15)
                eidl[pl.ds(woff, 16)] = pe
                dlocl[pl.ds(woff, 16)] = pd
                return (bw & (bw - 1), off2 + 1, pe, pd)

            _, off, pend_e, pend_d = lax.fori_loop(
                0, cnt, match, (bw0, off, pend_e, pend_d))
            return (off, pend_e, pend_d)

        zero16 = jnp.zeros((16,), jnp.int32)
        nm, _, _ = lax.fori_loop(0, SCC // 16, group, (0, zero16, zero16))

        def batch(b, c):
            bo = b * GB
            pltpu.async_copy(z_hbm.at[eidl.at[pl.ds(bo, GB)]], rows, sem).wait()
            rem = jnp.minimum(nm - bo, GB)

            def edge(r, c2):
                dl = dlocl[pl.ds(bo + r, 16)][0]
                for j in range(HH // 16):
                    s = pl.ds(j * 16, 16)
                    acc[dl, s] = jnp.maximum(acc[dl, s], rows[r, s])
                return c2

            lax.fori_loop(0, rem, edge, 0)
            return c

        lax.fori_loop(0, (nm + GB - 1) // GB, batch, 0)
        return carry

    lax.fori_loop(0, EE // SCC, chunk, 0)

    def finrow(r, c):
        for j in range(HH // 16):
            s = pl.ds(j * 16, 16)
            v = acc[r, s]
            acc[r, s] = jnp.where(v == -jnp.inf, 0.0, v)
        return c

    lax.fori_loop(0, RPW, finrow, 0)

    @pl.when(wid == NW - 1)
    def _():
        pltpu.sync_copy(acc.at[pl.ds(0, LAST_R)],
                        out_hbm.at[pl.ds(lo, LAST_R)])

    @pl.when(wid != NW - 1)
    def _():
        pltpu.sync_copy(acc, out_hbm.at[pl.ds(lo, RPW)])


def kernel(x, edge_index, W1, b1, W2, b2, W3, b3, W4, b4):
    src = edge_index[0]
    dst = edge_index[1]
    a1, bt1 = _tables(x, W1, b1.reshape(1, HH), relu_in=False)
    pre1 = _gather_sc(a1, bt1, src, dst)
    z1 = _zmat(pre1, W2, b2.reshape(1, HH))
    h = _scatter_sc(z1, dst)
    a2, bt2 = _tables(h, W3, b3.reshape(1, DD), relu_in=True)
    pre2 = _gather_sc(a2, bt2, src, dst)
    z2 = _zmat(pre2, W4, b4.reshape(1, DD))
    out = _scatter_sc(z2, dst)
    return out


# speedup vs baseline: 1.8654x; 1.8654x over previous
import functools

import jax
import jax.numpy as jnp
from jax import lax
from jax.experimental import pallas as pl
from jax.experimental.pallas import tpu as pltpu
from jax.experimental.pallas import tpu_sc as plsc

NN = 10000
EE = 320000
DD = 128
HH = 128

NW = 32
RPW = 320
LAST_R = NN - (NW - 1) * RPW
EPW = EE // NW
GC = 80
SCC = 4000
GB = 128

_mesh = plsc.VectorSubcoreMesh(core_axis_name="c", subcore_axis_name="s")


def _tables_body(relu_in, x_ref, w_ref, bias_ref, a_ref, b_ref):
    x = x_ref[...]
    if relu_in:
        x = jnp.maximum(x, 0.0)
    wt = w_ref[:DD, :]
    wb = w_ref[DD:, :]
    a_ref[...] = (
        jnp.dot(x, wt - wb, preferred_element_type=jnp.float32,
                precision=lax.Precision.HIGHEST)
        + bias_ref[...]
    )
    b_ref[...] = jnp.dot(x, wb, preferred_element_type=jnp.float32,
                         precision=lax.Precision.HIGHEST)


def _tables(x, w, bias, relu_in):
    blk = 1000
    return pl.pallas_call(
        functools.partial(_tables_body, relu_in),
        grid=(NN // blk,),
        in_specs=[
            pl.BlockSpec((blk, DD), lambda i: (i, 0)),
            pl.BlockSpec((2 * DD, HH), lambda i: (0, 0)),
            pl.BlockSpec((1, HH), lambda i: (0, 0)),
        ],
        out_specs=[
            pl.BlockSpec((blk, HH), lambda i: (i, 0)),
            pl.BlockSpec((blk, HH), lambda i: (i, 0)),
        ],
        out_shape=[
            jax.ShapeDtypeStruct((NN, HH), jnp.float32),
            jax.ShapeDtypeStruct((NN, HH), jnp.float32),
        ],
    )(x, w, bias)


def _zmat_body(p_ref, w_ref, bias_ref, z_ref):
    z_ref[...] = (
        jnp.dot(p_ref[...], w_ref[...], preferred_element_type=jnp.float32,
                precision=lax.Precision.HIGHEST)
        + bias_ref[...]
    )


def _zmat(pre, w, bias):
    blk = 2000
    return pl.pallas_call(
        _zmat_body,
        grid=(EE // blk,),
        in_specs=[
            pl.BlockSpec((blk, HH), lambda i: (i, 0)),
            pl.BlockSpec((HH, HH), lambda i: (0, 0)),
            pl.BlockSpec((1, HH), lambda i: (0, 0)),
        ],
        out_specs=pl.BlockSpec((blk, HH), lambda i: (i, 0)),
        out_shape=jax.ShapeDtypeStruct((EE, HH), jnp.float32),
    )(pre, w, bias)


@functools.partial(
    pl.kernel,
    out_type=jax.ShapeDtypeStruct((EE, HH), jnp.float32),
    mesh=_mesh,
    scratch_types=[
        pltpu.VMEM((GC,), jnp.int32),
        pltpu.VMEM((GC,), jnp.int32),
        pltpu.VMEM((GC, HH), jnp.float32),
        pltpu.VMEM((GC, HH), jnp.float32),
        pltpu.SemaphoreType.DMA,
    ],
)
def _gather_sc(a_hbm, b_hbm, src_hbm, dst_hbm, pre_hbm,
               dstv, srcv, rowsa, rowsb, sem):
    wid = lax.axis_index("s") * 2 + lax.axis_index("c")
    ebase = wid * EPW

    def chunk(i, carry):
        base = ebase + i * GC
        pltpu.sync_copy(dst_hbm.at[pl.ds(base, GC)], dstv)
        pltpu.sync_copy(src_hbm.at[pl.ds(base, GC)], srcv)
        cpa = pltpu.async_copy(a_hbm.at[dstv], rowsa, sem)
        cpb = pltpu.async_copy(b_hbm.at[srcv], rowsb, sem)
        cpa.wait()
        cpb.wait()

        def row(r, c):
            for j in range(HH // 16):
                s = pl.ds(j * 16, 16)
                rowsa[r, s] = jnp.maximum(rowsa[r, s] + rowsb[r, s], 0.0)
            return c

        lax.fori_loop(0, GC, row, 0)
        pltpu.sync_copy(rowsa, pre_hbm.at[pl.ds(base, GC)])
        return carry

    lax.fori_loop(0, EPW // GC, chunk, 0)


NCH = 25
CH = EPW // NCH
CAP = CH + 16

_DIVM = 52429


@functools.partial(
    pl.kernel,
    out_type=[
        jax.ShapeDtypeStruct((NW * NW * NCH * CAP,), jnp.int32),
        jax.ShapeDtypeStruct((NW * NW * NCH * CAP,), jnp.int32),
        jax.ShapeDtypeStruct((NW * NCH * NW,), jnp.int32),
    ],
    mesh=_mesh,
    scratch_types=[
        pltpu.VMEM((CH,), jnp.int32),
        pltpu.VMEM((NW * CAP,), jnp.int32),
        pltpu.VMEM((NW * CAP,), jnp.int32),
        pltpu.VMEM((NW,), jnp.int32),
        pltpu.SMEM((NW,), jnp.int32),
        pltpu.SemaphoreType.DMA,
    ],
)
def _bin_sc(dst_hbm, le_hbm, ld_hbm, cnt_hbm,
            dstv, lle, lld, cntv, offs, sem):
    wid = lax.axis_index("s") * 2 + lax.axis_index("c")
    ebase = wid * EPW
    iota = lax.iota(jnp.int32, 16)
    zero16 = jnp.zeros((16,), jnp.int32)

    def chunk(ch, carry):
        pltpu.sync_copy(dst_hbm.at[pl.ds(ebase + ch * CH, CH)], dstv)
        for o in range(NW):
            offs[o] = 0

        def group(g, c):
            d = dstv[pl.ds(g * 16, 16)]
            own16 = ((d >> 6) * _DIVM) >> 18
            dloc16 = d - own16 * RPW
            eb = ebase + ch * CH + g * 16
            for l in range(16):
                o = own16[l]
                off = offs[o]
                base = o * CAP + off
                lle[pl.ds(base, 16)] = jnp.full((16,), eb + l, jnp.int32)
                lld[pl.ds(base, 16)] = jnp.full((16,), dloc16[l], jnp.int32)
                offs[o] = off + 1
            return c

        lax.fori_loop(0, CH // 16, group, 0)

        c0 = zero16
        c1 = zero16
        cps = []
        for o in range(NW):
            c = offs[o]
            lle[pl.ds(o * CAP + c, 16)] = zero16
            lld[pl.ds(o * CAP + c, 16)] = zero16
            if o < 16:
                c0 = jnp.where(iota == o, c, c0)
            else:
                c1 = jnp.where(iota == (o - 16), c, c1)
            hoff = ((wid * NW + o) * NCH + ch) * CAP
            cps.append(pltpu.async_copy(lle.at[pl.ds(o * CAP, CAP)],
                                        le_hbm.at[pl.ds(hoff, CAP)], sem))
            cps.append(pltpu.async_copy(lld.at[pl.ds(o * CAP, CAP)],
                                        ld_hbm.at[pl.ds(hoff, CAP)], sem))
        cntv[pl.ds(0, 16)] = c0
        cntv[pl.ds(16, 16)] = c1
        pltpu.sync_copy(cntv, cnt_hbm.at[pl.ds((wid * NCH + ch) * NW, NW)])
        for cp in cps:
            cp.wait()
        return carry

    lax.fori_loop(0, NCH, chunk, 0)


LCAP = ((EPW + 16 + GB - 1) // GB) * GB


@functools.partial(
    pl.kernel,
    out_type=jax.ShapeDtypeStruct((NN, HH), jnp.float32),
    mesh=_mesh,
    scratch_types=[
        pltpu.VMEM((NCH * CAP,), jnp.int32),
        pltpu.VMEM((NCH * CAP,), jnp.int32),
        pltpu.VMEM((NCH * NW + 16,), jnp.int32),
        pltpu.VMEM((LCAP,), jnp.int32),
        pltpu.VMEM((LCAP,), jnp.int32),
        pltpu.VMEM((GB, HH), jnp.float32),
        pltpu.VMEM((RPW, HH), jnp.float32),
        pltpu.SemaphoreType.DMA,
    ],
)
def _scatter_sc(z_hbm, le_hbm, ld_hbm, cnt_hbm, out_hbm,
                ble, bld, bcnt, lloce, llocd, rows, acc, sem):
    wid = lax.axis_index("s") * 2 + lax.axis_index("c")
    lo = wid * RPW
    neg = jnp.full((16,), -jnp.inf, jnp.float32)
    zero16 = jnp.zeros((16,), jnp.int32)

    def initrow(r, c):
        for j in range(HH // 16):
            acc[r, pl.ds(j * 16, 16)] = neg
        return c

    lax.fori_loop(0, RPW, initrow, 0)

    def initloc(i, c):
        lloce[pl.ds(i * 16, 16)] = zero16
        return c

    lax.fori_loop(0, LCAP // 16, initloc, 0)

    def per_src(src, carry):
        boff = (src * NW + wid) * NCH * CAP
        cpa = pltpu.async_copy(le_hbm.at[pl.ds(boff, NCH * CAP)], ble, sem)
        cpb = pltpu.async_copy(ld_hbm.at[pl.ds(boff, NCH * CAP)], bld, sem)
        cpc = pltpu.async_copy(
            cnt_hbm.at[pl.ds(src * NCH * NW, NCH * NW)],
            bcnt.at[pl.ds(0, NCH * NW)], sem)
        cpa.wait()
        cpb.wait()
        cpc.wait()

        def compact(ch, woff):
            c = bcnt[pl.ds(ch * NW + wid, 16)][0]

            def copy16(k, c2):
                s = pl.ds(ch * CAP + k * 16, 16)
                lloce[pl.ds(woff + k * 16, 16)] = ble[s]
                llocd[pl.ds(woff + k * 16, 16)] = bld[s]
                return c2

            lax.fori_loop(0, (c + 15) >> 4, copy16, 0)
            return woff + c

        ntot = lax.fori_loop(0, NCH, compact, 0)
        lloce[pl.ds(ntot, 16)] = zero16
        llocd[pl.ds(ntot, 16)] = zero16

        def batch(b, c):
            bo = b * GB
            pltpu.async_copy(z_hbm.at[lloce.at[pl.ds(bo, GB)]], rows,
                             sem).wait()
            rem = jnp.minimum(ntot - bo, GB)

            def edge(r, c2):
                dl = llocd[pl.ds(bo + r, 16)][0]
                for j in range(HH // 16):
                    s = pl.ds(j * 16, 16)
                    acc[dl, s] = jnp.maximum(acc[dl, s], rows[r, s])
                return c2

            lax.fori_loop(0, rem, edge, 0)
            return c

        lax.fori_loop(0, (ntot + GB - 1) // GB, batch, 0)
        return carry

    lax.fori_loop(0, NW, per_src, 0)

    def finrow(r, c):
        for j in range(HH // 16):
            s = pl.ds(j * 16, 16)
            v = acc[r, s]
            acc[r, s] = jnp.where(v == -jnp.inf, 0.0, v)
        return c

    lax.fori_loop(0, RPW, finrow, 0)

    @pl.when(wid == NW - 1)
    def _():
        pltpu.sync_copy(acc.at[pl.ds(0, LAST_R)],
                        out_hbm.at[pl.ds(lo, LAST_R)])

    @pl.when(wid != NW - 1)
    def _():
        pltpu.sync_copy(acc, out_hbm.at[pl.ds(lo, RPW)])


def kernel(x, edge_index, W1, b1, W2, b2, W3, b3, W4, b4):
    src = edge_index[0]
    dst = edge_index[1]
    le, ld, cnt = _bin_sc(dst)
    a1, bt1 = _tables(x, W1, b1.reshape(1, HH), relu_in=False)
    pre1 = _gather_sc(a1, bt1, src, dst)
    z1 = _zmat(pre1, W2, b2.reshape(1, HH))
    h = _scatter_sc(z1, le, ld, cnt)
    a2, bt2 = _tables(h, W3, b3.reshape(1, DD), relu_in=True)
    pre2 = _gather_sc(a2, bt2, src, dst)
    z2 = _zmat(pre2, W4, b4.reshape(1, DD))
    out = _scatter_sc(z2, le, ld, cnt)
    return out


# speedup vs baseline: 1.9802x; 1.0615x over previous
import functools

import jax
import jax.numpy as jnp
from jax import lax
from jax.experimental import pallas as pl
from jax.experimental.pallas import tpu as pltpu
from jax.experimental.pallas import tpu_sc as plsc

NN = 10000
EE = 320000
DD = 128
HH = 128

NW = 32
RPW = 320
LAST_R = NN - (NW - 1) * RPW
EPW = EE // NW
GC = 400
GSUB = 80
SCC = 4000
GB = 128

_mesh = plsc.VectorSubcoreMesh(core_axis_name="c", subcore_axis_name="s")


def _tables_body(relu_in, x_ref, w_ref, bias_ref, a_ref, b_ref):
    x = x_ref[...]
    if relu_in:
        x = jnp.maximum(x, 0.0)
    wt = w_ref[:DD, :]
    wb = w_ref[DD:, :]
    a_ref[...] = (
        jnp.dot(x, wt - wb, preferred_element_type=jnp.float32,
                precision=lax.Precision.HIGHEST)
        + bias_ref[...]
    )
    b_ref[...] = jnp.dot(x, wb, preferred_element_type=jnp.float32,
                         precision=lax.Precision.HIGHEST)


def _tables(x, w, bias, relu_in):
    blk = 1000
    return pl.pallas_call(
        functools.partial(_tables_body, relu_in),
        grid=(NN // blk,),
        in_specs=[
            pl.BlockSpec((blk, DD), lambda i: (i, 0)),
            pl.BlockSpec((2 * DD, HH), lambda i: (0, 0)),
            pl.BlockSpec((1, HH), lambda i: (0, 0)),
        ],
        out_specs=[
            pl.BlockSpec((blk, HH), lambda i: (i, 0)),
            pl.BlockSpec((blk, HH), lambda i: (i, 0)),
        ],
        out_shape=[
            jax.ShapeDtypeStruct((NN, HH), jnp.float32),
            jax.ShapeDtypeStruct((NN, HH), jnp.float32),
        ],
    )(x, w, bias)


def _zmat_body(p_ref, w_ref, bias_ref, z_ref):
    z_ref[...] = (
        jnp.dot(p_ref[...], w_ref[...], preferred_element_type=jnp.float32,
                precision=lax.Precision.HIGHEST)
        + bias_ref[...]
    )


def _zmat(pre, w, bias):
    blk = 2000
    return pl.pallas_call(
        _zmat_body,
        grid=(EE // blk,),
        in_specs=[
            pl.BlockSpec((blk, HH), lambda i: (i, 0)),
            pl.BlockSpec((HH, HH), lambda i: (0, 0)),
            pl.BlockSpec((1, HH), lambda i: (0, 0)),
        ],
        out_specs=pl.BlockSpec((blk, HH), lambda i: (i, 0)),
        out_shape=jax.ShapeDtypeStruct((EE, HH), jnp.float32),
    )(pre, w, bias)


@functools.partial(
    pl.kernel,
    out_type=jax.ShapeDtypeStruct((EE, HH), jnp.float32),
    mesh=_mesh,
    scratch_types=[
        pltpu.VMEM((GC,), jnp.int32),
        pltpu.VMEM((GC,), jnp.int32),
        pltpu.VMEM((GC, HH), jnp.float32),
        pltpu.VMEM((GC, HH), jnp.float32),
        pltpu.SemaphoreType.DMA,
    ],
)
def _gather_sc(a_hbm, b_hbm, src_hbm, dst_hbm, pre_hbm,
               dstv, srcv, rowsa, rowsb, sem):
    wid = lax.axis_index("s") * 2 + lax.axis_index("c")
    ebase = wid * EPW

    def chunk(i, carry):
        base = ebase + i * GC
        pltpu.sync_copy(dst_hbm.at[pl.ds(base, GC)], dstv)
        pltpu.sync_copy(src_hbm.at[pl.ds(base, GC)], srcv)
        cps = []
        for k in range(GC // GSUB):
            so = pl.ds(k * GSUB, GSUB)
            cps.append(pltpu.async_copy(
                a_hbm.at[dstv.at[so]], rowsa.at[so], sem))
            cps.append(pltpu.async_copy(
                b_hbm.at[srcv.at[so]], rowsb.at[so], sem))
        for cp in cps:
            cp.wait()

        def row(r, c):
            avs = [rowsa[r, pl.ds(j * 16, 16)] for j in range(HH // 16)]
            bvs = [rowsb[r, pl.ds(j * 16, 16)] for j in range(HH // 16)]
            for j in range(HH // 16):
                rowsa[r, pl.ds(j * 16, 16)] = jnp.maximum(avs[j] + bvs[j], 0.0)
            return c

        lax.fori_loop(0, GC, row, 0)
        pltpu.sync_copy(rowsa, pre_hbm.at[pl.ds(base, GC)])
        return carry

    lax.fori_loop(0, EPW // GC, chunk, 0)


NCH = 25
CH = EPW // NCH
CAP = CH + 16

_DIVM = 52429


@functools.partial(
    pl.kernel,
    out_type=[
        jax.ShapeDtypeStruct((NW * NW * NCH * CAP,), jnp.int32),
        jax.ShapeDtypeStruct((NW * NW * NCH * CAP,), jnp.int32),
        jax.ShapeDtypeStruct((NW * NCH * NW,), jnp.int32),
    ],
    mesh=_mesh,
    scratch_types=[
        pltpu.VMEM((CH,), jnp.int32),
        pltpu.VMEM((NW * CAP,), jnp.int32),
        pltpu.VMEM((NW * CAP,), jnp.int32),
        pltpu.VMEM((NW,), jnp.int32),
        pltpu.SMEM((NW,), jnp.int32),
        pltpu.SemaphoreType.DMA,
    ],
)
def _bin_sc(dst_hbm, le_hbm, ld_hbm, cnt_hbm,
            dstv, lle, lld, cntv, offs, sem):
    wid = lax.axis_index("s") * 2 + lax.axis_index("c")
    ebase = wid * EPW
    iota = lax.iota(jnp.int32, 16)
    zero16 = jnp.zeros((16,), jnp.int32)

    def chunk(ch, carry):
        pltpu.sync_copy(dst_hbm.at[pl.ds(ebase + ch * CH, CH)], dstv)
        for o in range(NW):
            offs[o] = 0

        def group(g, c):
            d = dstv[pl.ds(g * 16, 16)]
            own16 = ((d >> 6) * _DIVM) >> 18
            dloc16 = d - own16 * RPW
            eb = ebase + ch * CH + g * 16
            for l in range(16):
                o = own16[l]
                off = offs[o]
                base = o * CAP + off
                lle[pl.ds(base, 16)] = jnp.full((16,), eb + l, jnp.int32)
                lld[pl.ds(base, 16)] = jnp.full((16,), dloc16[l], jnp.int32)
                offs[o] = off + 1
            return c

        lax.fori_loop(0, CH // 16, group, 0)

        c0 = zero16
        c1 = zero16
        cps = []
        for o in range(NW):
            c = offs[o]
            lle[pl.ds(o * CAP + c, 16)] = zero16
            lld[pl.ds(o * CAP + c, 16)] = zero16
            if o < 16:
                c0 = jnp.where(iota == o, c, c0)
            else:
                c1 = jnp.where(iota == (o - 16), c, c1)
            hoff = ((wid * NW + o) * NCH + ch) * CAP
            cps.append(pltpu.async_copy(lle.at[pl.ds(o * CAP, CAP)],
                                        le_hbm.at[pl.ds(hoff, CAP)], sem))
            cps.append(pltpu.async_copy(lld.at[pl.ds(o * CAP, CAP)],
                                        ld_hbm.at[pl.ds(hoff, CAP)], sem))
        cntv[pl.ds(0, 16)] = c0
        cntv[pl.ds(16, 16)] = c1
        pltpu.sync_copy(cntv, cnt_hbm.at[pl.ds((wid * NCH + ch) * NW, NW)])
        for cp in cps:
            cp.wait()
        return carry

    lax.fori_loop(0, NCH, chunk, 0)


LCAP = ((EPW + 16 + GB - 1) // GB) * GB


@functools.partial(
    pl.kernel,
    out_type=jax.ShapeDtypeStruct((NN, HH), jnp.float32),
    mesh=_mesh,
    scratch_types=[
        pltpu.VMEM((NCH * CAP,), jnp.int32),
        pltpu.VMEM((NCH * CAP,), jnp.int32),
        pltpu.VMEM((NCH * NW + 16,), jnp.int32),
        pltpu.VMEM((LCAP,), jnp.int32),
        pltpu.VMEM((LCAP,), jnp.int32),
        pltpu.VMEM((GB, HH), jnp.float32),
        pltpu.VMEM((RPW, HH), jnp.float32),
        pltpu.SemaphoreType.DMA,
    ],
)
def _scatter_sc(z_hbm, le_hbm, ld_hbm, cnt_hbm, out_hbm,
                ble, bld, bcnt, lloce, llocd, rows, acc, sem):
    wid = lax.axis_index("s") * 2 + lax.axis_index("c")
    lo = wid * RPW
    neg = jnp.full((16,), -jnp.inf, jnp.float32)
    zero16 = jnp.zeros((16,), jnp.int32)

    def initrow(r, c):
        for j in range(HH // 16):
            acc[r, pl.ds(j * 16, 16)] = neg
        return c

    lax.fori_loop(0, RPW, initrow, 0)

    def initloc(i, c):
        lloce[pl.ds(i * 16, 16)] = zero16
        return c

    lax.fori_loop(0, LCAP // 16, initloc, 0)

    def per_src(src, carry):
        boff = (src * NW + wid) * NCH * CAP
        cpa = pltpu.async_copy(le_hbm.at[pl.ds(boff, NCH * CAP)], ble, sem)
        cpb = pltpu.async_copy(ld_hbm.at[pl.ds(boff, NCH * CAP)], bld, sem)
        cpc = pltpu.async_copy(
            cnt_hbm.at[pl.ds(src * NCH * NW, NCH * NW)],
            bcnt.at[pl.ds(0, NCH * NW)], sem)
        cpa.wait()
        cpb.wait()
        cpc.wait()

        def compact(ch, woff):
            c = bcnt[pl.ds(ch * NW + wid, 16)][0]

            def copy16(k, c2):
                s = pl.ds(ch * CAP + k * 16, 16)
                lloce[pl.ds(woff + k * 16, 16)] = ble[s]
                llocd[pl.ds(woff + k * 16, 16)] = bld[s]
                return c2

            lax.fori_loop(0, (c + 15) >> 4, copy16, 0)
            return woff + c

        ntot = lax.fori_loop(0, NCH, compact, 0)
        lloce[pl.ds(ntot, 16)] = zero16
        llocd[pl.ds(ntot, 16)] = zero16

        def batch(b, c):
            bo = b * GB
            pltpu.async_copy(z_hbm.at[lloce.at[pl.ds(bo, GB)]], rows,
                             sem).wait()
            rem = jnp.minimum(ntot - bo, GB)

            def edge(r, c2):
                dl = llocd[pl.ds(bo + r, 16)][0]
                avs = [acc[dl, pl.ds(j * 16, 16)] for j in range(HH // 16)]
                rvs = [rows[r, pl.ds(j * 16, 16)] for j in range(HH // 16)]
                for j in range(HH // 16):
                    acc[dl, pl.ds(j * 16, 16)] = jnp.maximum(avs[j], rvs[j])
                return c2

            lax.fori_loop(0, rem, edge, 0)
            return c

        lax.fori_loop(0, (ntot + GB - 1) // GB, batch, 0)
        return carry

    lax.fori_loop(0, NW, per_src, 0)

    def finrow(r, c):
        for j in range(HH // 16):
            s = pl.ds(j * 16, 16)
            v = acc[r, s]
            acc[r, s] = jnp.where(v == -jnp.inf, 0.0, v)
        return c

    lax.fori_loop(0, RPW, finrow, 0)

    @pl.when(wid == NW - 1)
    def _():
        pltpu.sync_copy(acc.at[pl.ds(0, LAST_R)],
                        out_hbm.at[pl.ds(lo, LAST_R)])

    @pl.when(wid != NW - 1)
    def _():
        pltpu.sync_copy(acc, out_hbm.at[pl.ds(lo, RPW)])


def kernel(x, edge_index, W1, b1, W2, b2, W3, b3, W4, b4):
    src = edge_index[0]
    dst = edge_index[1]
    le, ld, cnt = _bin_sc(dst)
    a1, bt1 = _tables(x, W1, b1.reshape(1, HH), relu_in=False)
    pre1 = _gather_sc(a1, bt1, src, dst)
    z1 = _zmat(pre1, W2, b2.reshape(1, HH))
    h = _scatter_sc(z1, le, ld, cnt)
    a2, bt2 = _tables(h, W3, b3.reshape(1, DD), relu_in=True)
    pre2 = _gather_sc(a2, bt2, src, dst)
    z2 = _zmat(pre2, W4, b4.reshape(1, DD))
    out = _scatter_sc(z2, le, ld, cnt)
    return out


# speedup vs baseline: 3.0379x; 1.5341x over previous
import functools

import jax
import jax.numpy as jnp
from jax import lax
from jax.experimental import pallas as pl
from jax.experimental.pallas import tpu as pltpu
from jax.experimental.pallas import tpu_sc as plsc

NN = 10000
EE = 320000
DD = 128
HH = 128

NW = 32
RPW = 320
LAST_R = NN - (NW - 1) * RPW
EPW = EE // NW
GC = 400
GSUB = 80
SCC = 4000
GB = 128

_mesh = plsc.VectorSubcoreMesh(core_axis_name="c", subcore_axis_name="s")


def _tables_body(relu_in, x_ref, w_ref, bias_ref, a_ref, b_ref):
    x = x_ref[...]
    if relu_in:
        x = jnp.maximum(x, 0.0)
    wt = w_ref[:DD, :]
    wb = w_ref[DD:, :]
    a_ref[...] = (
        jnp.dot(x, wt - wb, preferred_element_type=jnp.float32,
                precision=lax.Precision.HIGHEST)
        + bias_ref[...]
    )
    b_ref[...] = jnp.dot(x, wb, preferred_element_type=jnp.float32,
                         precision=lax.Precision.HIGHEST)


def _tables(x, w, bias, relu_in):
    blk = 1000
    return pl.pallas_call(
        functools.partial(_tables_body, relu_in),
        grid=(NN // blk,),
        in_specs=[
            pl.BlockSpec((blk, DD), lambda i: (i, 0)),
            pl.BlockSpec((2 * DD, HH), lambda i: (0, 0)),
            pl.BlockSpec((1, HH), lambda i: (0, 0)),
        ],
        out_specs=[
            pl.BlockSpec((blk, HH), lambda i: (i, 0)),
            pl.BlockSpec((blk, HH), lambda i: (i, 0)),
        ],
        out_shape=[
            jax.ShapeDtypeStruct((NN, HH), jnp.float32),
            jax.ShapeDtypeStruct((NN, HH), jnp.float32),
        ],
    )(x, w, bias)


def _zmat_body(p_ref, w_ref, bias_ref, z_ref):
    z_ref[...] = (
        jnp.dot(p_ref[...], w_ref[...], preferred_element_type=jnp.float32,
                precision=lax.Precision.HIGHEST)
        + bias_ref[...]
    )


def _zmat(pre, w, bias):
    blk = 2000
    return pl.pallas_call(
        _zmat_body,
        grid=(EE // blk,),
        in_specs=[
            pl.BlockSpec((blk, HH), lambda i: (i, 0)),
            pl.BlockSpec((HH, HH), lambda i: (0, 0)),
            pl.BlockSpec((1, HH), lambda i: (0, 0)),
        ],
        out_specs=pl.BlockSpec((blk, HH), lambda i: (i, 0)),
        out_shape=jax.ShapeDtypeStruct((EE, HH), jnp.float32),
    )(pre, w, bias)


@functools.partial(
    pl.kernel,
    out_type=jax.ShapeDtypeStruct((EE, HH), jnp.float32),
    mesh=_mesh,
    scratch_types=[
        pltpu.VMEM((GC,), jnp.int32),
        pltpu.VMEM((GC,), jnp.int32),
        pltpu.VMEM((GC, HH), jnp.float32),
        pltpu.VMEM((GC, HH), jnp.float32),
        pltpu.SemaphoreType.DMA,
    ],
)
def _gather_sc(a_hbm, b_hbm, src_hbm, dst_hbm, pre_hbm,
               dstv, srcv, rowsa, rowsb, sem):
    wid = lax.axis_index("s") * 2 + lax.axis_index("c")
    ebase = wid * EPW

    def chunk(i, carry):
        base = ebase + i * GC
        pltpu.sync_copy(dst_hbm.at[pl.ds(base, GC)], dstv)
        pltpu.sync_copy(src_hbm.at[pl.ds(base, GC)], srcv)
        cps = []
        for k in range(GC // GSUB):
            so = pl.ds(k * GSUB, GSUB)
            cps.append(pltpu.async_copy(
                a_hbm.at[dstv.at[so]], rowsa.at[so], sem))
            cps.append(pltpu.async_copy(
                b_hbm.at[srcv.at[so]], rowsb.at[so], sem))
        for cp in cps:
            cp.wait()

        def row(r, c):
            avs = [rowsa[r, pl.ds(j * 16, 16)] for j in range(HH // 16)]
            bvs = [rowsb[r, pl.ds(j * 16, 16)] for j in range(HH // 16)]
            for j in range(HH // 16):
                rowsa[r, pl.ds(j * 16, 16)] = jnp.maximum(avs[j] + bvs[j], 0.0)
            return c

        lax.fori_loop(0, GC, row, 0)
        pltpu.sync_copy(rowsa, pre_hbm.at[pl.ds(base, GC)])
        return carry

    lax.fori_loop(0, EPW // GC, chunk, 0)


NCH = 25
CH = EPW // NCH
CAP = CH + 16

_DIVM = 52429


@functools.partial(
    pl.kernel,
    out_type=[
        jax.ShapeDtypeStruct((NW * NW * NCH * CAP,), jnp.int32),
        jax.ShapeDtypeStruct((NW * NW * NCH * CAP,), jnp.int32),
        jax.ShapeDtypeStruct((NW * NCH * NW,), jnp.int32),
    ],
    mesh=_mesh,
    scratch_types=[
        pltpu.VMEM((CH,), jnp.int32),
        pltpu.VMEM((NW * CAP,), jnp.int32),
        pltpu.VMEM((NW * CAP,), jnp.int32),
        pltpu.VMEM((NW,), jnp.int32),
        pltpu.SMEM((NW,), jnp.int32),
        pltpu.SemaphoreType.DMA,
    ],
)
def _bin_sc(dst_hbm, le_hbm, ld_hbm, cnt_hbm,
            dstv, lle, lld, cntv, offs, sem):
    wid = lax.axis_index("s") * 2 + lax.axis_index("c")
    ebase = wid * EPW
    iota = lax.iota(jnp.int32, 16)
    zero16 = jnp.zeros((16,), jnp.int32)

    def chunk(ch, carry):
        pltpu.sync_copy(dst_hbm.at[pl.ds(ebase + ch * CH, CH)], dstv)
        for o in range(NW):
            offs[o] = 0

        def group(g, c):
            d = dstv[pl.ds(g * 16, 16)]
            own16 = ((d >> 6) * _DIVM) >> 18
            dloc16 = d - own16 * RPW
            eb = ebase + ch * CH + g * 16
            for l in range(16):
                o = own16[l]
                off = offs[o]
                base = o * CAP + off
                lle[pl.ds(base, 16)] = jnp.full((16,), eb + l, jnp.int32)
                lld[pl.ds(base, 16)] = jnp.full((16,), dloc16[l], jnp.int32)
                offs[o] = off + 1
            return c

        lax.fori_loop(0, CH // 16, group, 0)

        c0 = zero16
        c1 = zero16
        rpw16 = jnp.full((16,), RPW, jnp.int32)
        cps = []
        for o in range(NW):
            c = offs[o]
            lle[pl.ds(o * CAP + c, 16)] = zero16
            lld[pl.ds(o * CAP + c, 16)] = rpw16
            if o < 16:
                c0 = jnp.where(iota == o, c, c0)
            else:
                c1 = jnp.where(iota == (o - 16), c, c1)
            hoff = ((wid * NW + o) * NCH + ch) * CAP
            cps.append(pltpu.async_copy(lle.at[pl.ds(o * CAP, CAP)],
                                        le_hbm.at[pl.ds(hoff, CAP)], sem))
            cps.append(pltpu.async_copy(lld.at[pl.ds(o * CAP, CAP)],
                                        ld_hbm.at[pl.ds(hoff, CAP)], sem))
        cntv[pl.ds(0, 16)] = c0
        cntv[pl.ds(16, 16)] = c1
        pltpu.sync_copy(cntv, cnt_hbm.at[pl.ds((wid * NCH + ch) * NW, NW)])
        for cp in cps:
            cp.wait()
        return carry

    lax.fori_loop(0, NCH, chunk, 0)


GB2 = 64
LCAP = ((EPW + 16 + GB2 - 1) // GB2) * GB2


@functools.partial(
    pl.kernel,
    out_type=jax.ShapeDtypeStruct((NN, HH), jnp.float32),
    mesh=_mesh,
    scratch_types=[
        pltpu.VMEM((2 * NCH * CAP,), jnp.int32),
        pltpu.VMEM((2 * NCH * CAP,), jnp.int32),
        pltpu.VMEM((2 * (NCH * NW + 16),), jnp.int32),
        pltpu.VMEM((LCAP,), jnp.int32),
        pltpu.VMEM((LCAP,), jnp.int32),
        pltpu.VMEM((2 * GB2, HH), jnp.float32),
        pltpu.VMEM((RPW + 1, HH), jnp.float32),
        pltpu.SemaphoreType.DMA,
        pltpu.SemaphoreType.DMA,
        pltpu.SemaphoreType.DMA,
        pltpu.SemaphoreType.DMA,
    ],
)
def _scatter_sc(z_hbm, le_hbm, ld_hbm, cnt_hbm, out_hbm,
                ble, bld, bcnt, lloce, llocd, rows, acc,
                semb0, semb1, sem0, sem1):
    wid = lax.axis_index("s") * 2 + lax.axis_index("c")
    lo = wid * RPW
    neg = jnp.full((16,), -jnp.inf, jnp.float32)
    zero16 = jnp.zeros((16,), jnp.int32)
    rpw16 = jnp.full((16,), RPW, jnp.int32)

    def initrow(r, c):
        for j in range(HH // 16):
            acc[r, pl.ds(j * 16, 16)] = neg
        return c

    lax.fori_loop(0, RPW + 1, initrow, 0)

    def initloc(i, c):
        lloce[pl.ds(i * 16, 16)] = zero16
        llocd[pl.ds(i * 16, 16)] = rpw16
        return c

    lax.fori_loop(0, LCAP // 16, initloc, 0)

    CNTS = NCH * NW + 16

    def fire_blocks(s, p, semb):
        boff = (s * NW + wid) * NCH * CAP
        pltpu.async_copy(le_hbm.at[pl.ds(boff, NCH * CAP)],
                         ble.at[pl.ds(p * NCH * CAP, NCH * CAP)], semb)
        pltpu.async_copy(ld_hbm.at[pl.ds(boff, NCH * CAP)],
                         bld.at[pl.ds(p * NCH * CAP, NCH * CAP)], semb)
        pltpu.async_copy(cnt_hbm.at[pl.ds(s * NCH * NW, NCH * NW)],
                         bcnt.at[pl.ds(p * CNTS, NCH * NW)], semb)

    def drain_blocks(p, semb):
        pltpu.make_async_copy(
            le_hbm.at[pl.ds(0, NCH * CAP)],
            ble.at[pl.ds(p * NCH * CAP, NCH * CAP)], semb).wait()
        pltpu.make_async_copy(
            ld_hbm.at[pl.ds(0, NCH * CAP)],
            bld.at[pl.ds(p * NCH * CAP, NCH * CAP)], semb).wait()
        pltpu.make_async_copy(
            cnt_hbm.at[pl.ds(0, NCH * NW)],
            bcnt.at[pl.ds(p * CNTS, NCH * NW)], semb).wait()

    def process_src(p):
        def compact(ch, woff):
            c = bcnt[pl.ds(p * CNTS + ch * NW + wid, 16)][0]

            def copy16(k, c2):
                s = pl.ds(p * NCH * CAP + ch * CAP + k * 16, 16)
                lloce[pl.ds(woff + k * 16, 16)] = ble[s]
                llocd[pl.ds(woff + k * 16, 16)] = bld[s]
                return c2

            lax.fori_loop(0, (c + 15) >> 4, copy16, 0)
            return woff + c

        ntot = lax.fori_loop(0, NCH, compact, 0)
        lloce[pl.ds(ntot, 16)] = zero16
        llocd[pl.ds(ntot, 16)] = rpw16

        nb = jnp.maximum((ntot + GB2 - 1) // GB2, 1)
        pltpu.async_copy(z_hbm.at[lloce.at[pl.ds(0, GB2)]],
                         rows.at[pl.ds(0, GB2)], sem0)

        @pl.when(nb > 1)
        def _():
            pltpu.async_copy(z_hbm.at[lloce.at[pl.ds(GB2, GB2)]],
                             rows.at[pl.ds(GB2, GB2)], sem1)

        def batch(b, c):
            def half(q, sq):
                pltpu.make_async_copy(
                    z_hbm.at[pl.ds(0, GB2)],
                    rows.at[pl.ds(q * GB2, GB2)], sq).wait()

                bo = b * GB2
                ne = jnp.minimum(ntot - bo, GB2)

                def edge4(r4, c2):
                    for u in range(4):
                        r = r4 * 4 + u
                        dl = llocd[pl.ds(bo + r, 16)][0]
                        avs = [acc[dl, pl.ds(j * 16, 16)]
                               for j in range(HH // 16)]
                        rvs = [rows[q * GB2 + r, pl.ds(j * 16, 16)]
                               for j in range(HH // 16)]
                        for j in range(HH // 16):
                            acc[dl, pl.ds(j * 16, 16)] = jnp.maximum(
                                avs[j], rvs[j])
                    return c2

                lax.fori_loop(0, (jnp.maximum(ne, 0) + 3) >> 2, edge4, 0)

                @pl.when(b + 2 < nb)
                def _():
                    pltpu.async_copy(
                        z_hbm.at[lloce.at[pl.ds((b + 2) * GB2, GB2)]],
                        rows.at[pl.ds(q * GB2, GB2)], sq)

            @pl.when(b % 2 == 0)
            def _():
                half(0, sem0)

            @pl.when(b % 2 == 1)
            def _():
                half(1, sem1)

            return c

        lax.fori_loop(0, nb, batch, 0)

    fire_blocks(0, 0, semb0)

    def src_pair(i, carry):
        s0 = 2 * i
        drain_blocks(0, semb0)
        fire_blocks(s0 + 1, 1, semb1)
        process_src(0)
        drain_blocks(1, semb1)

        @pl.when(s0 + 2 < NW)
        def _():
            fire_blocks(s0 + 2, 0, semb0)

        process_src(1)
        return carry

    lax.fori_loop(0, NW // 2, src_pair, 0)

    def finrow(r, c):
        for j in range(HH // 16):
            s = pl.ds(j * 16, 16)
            v = acc[r, s]
            acc[r, s] = jnp.where(v == -jnp.inf, 0.0, v)
        return c

    lax.fori_loop(0, RPW, finrow, 0)

    @pl.when(wid == NW - 1)
    def _():
        pltpu.sync_copy(acc.at[pl.ds(0, LAST_R)],
                        out_hbm.at[pl.ds(lo, LAST_R)])

    @pl.when(wid != NW - 1)
    def _():
        pltpu.sync_copy(acc.at[pl.ds(0, RPW)], out_hbm.at[pl.ds(lo, RPW)])


def kernel(x, edge_index, W1, b1, W2, b2, W3, b3, W4, b4):
    src = edge_index[0]
    dst = edge_index[1]
    le, ld, cnt = _bin_sc(dst)
    a1, bt1 = _tables(x, W1, b1.reshape(1, HH), relu_in=False)
    pre1 = _gather_sc(a1, bt1, src, dst)
    z1 = _zmat(pre1, W2, b2.reshape(1, HH))
    h = _scatter_sc(z1, le, ld, cnt)
    a2, bt2 = _tables(h, W3, b3.reshape(1, DD), relu_in=True)
    pre2 = _gather_sc(a2, bt2, src, dst)
    z2 = _zmat(pre2, W4, b4.reshape(1, DD))
    out = _scatter_sc(z2, le, ld, cnt)
    return out
